# Initial kernel scaffold; baseline (speedup 1.0000x reference)
import jax
import jax.numpy as jnp
from jax.experimental import pallas as pl


def kernel(x, edge_index, Wl1, bl1, Wr1, Wl2, bl2, Wr2):
    raise NotImplementedError("write your pallas kernel here")


# speedup vs baseline: 32.0690x; 32.0690x over previous
import functools

import jax
import jax.numpy as jnp
from jax import lax
from jax.experimental import pallas as pl
from jax.experimental.pallas import tpu as pltpu
from jax.experimental.pallas import tpu_sc as plsc

N_NODES = 100000
N_EDGES = 3200000

NC = 2
NS = 16
NW = NC * NS
C = 128
IB = 16
CPW = 784
NBLK = CPW // IB
EPW = CPW * C
EPAD = NW * EPW
NPAD = 100352
RPT = NPAD // NS
TCB = 2048


def _make_edge_pass(width):
    mesh = plsc.VectorSubcoreMesh(core_axis_name="c", subcore_axis_name="s")

    @functools.partial(
        pl.kernel,
        out_type=jax.ShapeDtypeStruct((NC, NPAD, width), jnp.float32),
        mesh=mesh,
        compiler_params=pltpu.CompilerParams(use_tc_tiling_on_sc=False),
        scratch_types=[
            pltpu.VMEM_SHARED((NPAD, width), jnp.float32),
            pltpu.VMEM_SHARED((NPAD, width), jnp.float32),
            pltpu.VMEM((IB, C), jnp.int32),
            pltpu.VMEM((IB, C), jnp.int32),
            pltpu.VMEM((IB, C, width), jnp.float32),
        ],
    )
    def edge_pass(tab_hbm, src_hbm, dst_hbm, zero_hbm, out_hbm,
                  tab_sp, acc_sp, sbuf, dbuf, rows):
        c = lax.axis_index("c")
        s = lax.axis_index("s")
        w = c * NS + s
        r0 = s * RPT
        pltpu.sync_copy(tab_hbm.at[pl.ds(r0, RPT)], tab_sp.at[pl.ds(r0, RPT)])
        pltpu.sync_copy(zero_hbm.at[pl.ds(r0, RPT)], acc_sp.at[pl.ds(r0, RPT)])
        plsc.subcore_barrier()

        row0 = w * CPW

        def body(j, carry):
            rr = row0 + j * IB
            pltpu.sync_copy(src_hbm.at[pl.ds(rr, IB)], sbuf)
            pltpu.sync_copy(dst_hbm.at[pl.ds(rr, IB)], dbuf)
            for t in range(IB):
                pltpu.sync_copy(tab_sp.at[sbuf.at[t]], rows.at[t])
                pltpu.sync_copy(rows.at[t], acc_sp.at[dbuf.at[t]], add=True)
            return carry

        lax.fori_loop(0, NBLK, body, 0)
        plsc.subcore_barrier()
        pltpu.sync_copy(acc_sp.at[pl.ds(r0, RPT)], out_hbm.at[c, pl.ds(r0, RPT)])

    return edge_pass


_edge_pass4 = _make_edge_pass(4)
_edge_pass1 = _make_edge_pass(1)


def _tc1_body(p_ref, xT_ref, wl1_ref, bl1_ref, wr1_ref, wl2_ref, wr2_ref,
              bl2_ref, z_ref, r2_ref, deg_ref):
    a0 = p_ref[0, 0, :] + p_ref[1, 0, :]
    a1 = p_ref[0, 1, :] + p_ref[1, 1, :]
    d = p_ref[0, 2, :] + p_ref[1, 2, :]
    dc = jnp.maximum(d, 1.0)
    m0 = a0 / dc
    m1 = a1 / dc
    x0 = xT_ref[0, :]
    x1 = xT_ref[1, :]
    z = jnp.zeros_like(m0)
    r2 = jnp.zeros_like(m0)
    for f in range(16):
        h = jnp.maximum(
            m0 * wl1_ref[f, 0] + m1 * wl1_ref[f, 1] + bl1_ref[f]
            + x0 * wr1_ref[f, 0] + x1 * wr1_ref[f, 1], 0.0)
        z = z + h * wl2_ref[0, f]
        r2 = r2 + h * wr2_ref[0, f]
    z_ref[:] = z
    r2_ref[:] = r2 + bl2_ref[0]
    deg_ref[:] = dc


def _tc1(pT, xT, Wl1, bl1, Wr1, Wl2, Wr2, bl2):
    grid = NPAD // TCB
    smem = pl.BlockSpec(memory_space=pltpu.SMEM)
    return pl.pallas_call(
        _tc1_body,
        grid=(grid,),
        in_specs=[
            pl.BlockSpec((NC, 4, TCB), lambda i: (0, 0, i)),
            pl.BlockSpec((NC, TCB), lambda i: (0, i)),
            smem, smem, smem, smem, smem, smem,
        ],
        out_specs=[
            pl.BlockSpec((TCB,), lambda i: (i,)),
            pl.BlockSpec((TCB,), lambda i: (i,)),
            pl.BlockSpec((TCB,), lambda i: (i,)),
        ],
        out_shape=[
            jax.ShapeDtypeStruct((NPAD,), jnp.float32),
            jax.ShapeDtypeStruct((NPAD,), jnp.float32),
            jax.ShapeDtypeStruct((NPAD,), jnp.float32),
        ],
    )(pT, xT, Wl1, bl1, Wr1, Wl2, Wr2, bl2)


def _tc2_body(zp_ref, deg_ref, r2_ref, o_ref):
    zm = (zp_ref[0, :] + zp_ref[1, :]) / deg_ref[:]
    o_ref[:] = jax.nn.sigmoid(zm + r2_ref[:])


def _tc2(zp, degc, r2):
    grid = NPAD // TCB
    return pl.pallas_call(
        _tc2_body,
        grid=(grid,),
        in_specs=[
            pl.BlockSpec((NC, TCB), lambda i: (0, i)),
            pl.BlockSpec((TCB,), lambda i: (i,)),
            pl.BlockSpec((TCB,), lambda i: (i,)),
        ],
        out_specs=pl.BlockSpec((TCB,), lambda i: (i,)),
        out_shape=jax.ShapeDtypeStruct((NPAD,), jnp.float32),
    )(zp, degc, r2)


def kernel(x, edge_index, Wl1, bl1, Wr1, Wl2, bl2, Wr2):
    n = x.shape[0]
    e = edge_index.shape[1]
    src = edge_index[0].astype(jnp.int32)
    dst = edge_index[1].astype(jnp.int32)
    pad = EPAD - e
    src2d = jnp.concatenate(
        [src, jnp.full((pad,), n, jnp.int32)]).reshape(EPAD // C, C)
    dst2d = jnp.concatenate(
        [dst, jnp.full((pad,), n, jnp.int32)]).reshape(EPAD // C, C)

    xtab = jnp.zeros((NPAD, 4), jnp.float32)
    xtab = xtab.at[:n, 0:2].set(x).at[:n, 2].set(1.0)
    zeros4 = jnp.zeros((NPAD, 4), jnp.float32)

    partA = _edge_pass4(xtab, src2d, dst2d, zeros4)
    pT = jnp.transpose(partA, (0, 2, 1))
    xT = jnp.zeros((NC, NPAD), jnp.float32).at[:, :n].set(x.T)

    z, r2, degc = _tc1(pT, xT, Wl1, bl1, Wr1, Wl2, Wr2, bl2)

    ztab = z.reshape(NPAD, 1)
    zeros1 = jnp.zeros((NPAD, 1), jnp.float32)
    partB = _edge_pass1(ztab, src2d, dst2d, zeros1)

    out = _tc2(partB.reshape(NC, NPAD), degc, r2)
    return out[:n]


# speedup vs baseline: 84.7030x; 2.6413x over previous
import functools

import jax
import jax.numpy as jnp
from jax import lax
from jax.experimental import pallas as pl
from jax.experimental.pallas import tpu as pltpu
from jax.experimental.pallas import tpu_sc as plsc

NC = 2
NS = 16
NW = NC * NS
C = 128
IB = 16
NPAD = 100352
RPT = NPAD // NS
TCB = 2048
N = 100000
E = 3200000
NCH = E // C
BASE = NCH // NW
EXTRA = NCH - BASE * NW
NBLK = (BASE + IB) // IB
DUMMY = N
LASTR = N - (NS - 1) * RPT


def _edge_body(width, tabs_hbm, e_hbm, zacc_hbm, accs_hbm,
               tabs_sp, accs_sp, deg_sp, sbuf, dbuf, rowbufs, ones,
               semg, sems):
    c = lax.axis_index("c")
    s = lax.axis_index("s")
    w = c * NS + s
    r0 = s * RPT

    for tab_hbm, tab_sp in zip(tabs_hbm, tabs_sp):
        @pl.when(s < NS - 1)
        def _():
            pltpu.sync_copy(tab_hbm.at[pl.ds(r0, RPT)],
                            tab_sp.at[pl.ds(r0, RPT)])

        @pl.when(s == NS - 1)
        def _():
            pltpu.sync_copy(tab_hbm.at[pl.ds(r0, LASTR)],
                            tab_sp.at[pl.ds(r0, LASTR)])

    for acc_sp, _ in accs_sp:
        pltpu.sync_copy(zacc_hbm.at[pl.ds(r0, RPT)], acc_sp.at[pl.ds(r0, RPT)])
    if deg_sp is not None:
        pltpu.sync_copy(zacc_hbm.at[pl.ds(r0, RPT)], deg_sp.at[pl.ds(r0, RPT)])
        for g in range(C // 16):
            ones[pl.ds(g * 16, 16)] = jnp.full((16,), 1.0, jnp.float32)
    plsc.subcore_barrier()

    start = w * BASE + jnp.minimum(w, EXTRA)
    count = BASE + jnp.where(w < EXTRA, 1, 0).astype(jnp.int32)
    maxoff = NCH - IB

    def off_of(j):
        return jnp.minimum(start + j * IB, maxoff)

    def body(j, carry):
        off = off_of(j)
        pltpu.sync_copy(e_hbm.at[0, pl.ds(off, IB)], sbuf)
        pltpu.sync_copy(e_hbm.at[1, pl.ds(off, IB)], dbuf)
        @pl.when(j == NBLK - 1)
        def _():
            for t in range(IB):
                g = off + t
                bad = jnp.logical_or(g < start + (NBLK - 1) * IB,
                                     g >= start + count)

                @pl.when(bad)
                def _():
                    for gr in range(C // 16):
                        dbuf[t, pl.ds(gr * 16, 16)] = jnp.full(
                            (16,), DUMMY, jnp.int32)

        gds = [pltpu.async_copy(tab_sp.at[sbuf.at[t]], rows.at[t], semg)
               for tab_sp, rows in zip(tabs_sp, rowbufs)
               for t in range(IB)]
        for dsc in gds:
            dsc.wait()
        sds = [pltpu.async_copy(rows.at[t], acc_sp.at[dbuf.at[t]], sems,
                                add=True)
               for acc_sp, rows in accs_sp
               for t in range(IB)]
        if deg_sp is not None:
            sds += [pltpu.async_copy(ones, deg_sp.at[dbuf.at[t]], sems,
                                     add=True)
                    for t in range(IB)]
        for dsc in sds:
            dsc.wait()
        return carry

    lax.fori_loop(0, NBLK, body, 0)
    plsc.subcore_barrier()
    for (acc_sp, out_hbm) in accs_hbm:
        pltpu.sync_copy(acc_sp.at[pl.ds(r0, RPT)],
                        out_hbm.at[c, pl.ds(r0, RPT)])


def _make_pass_a():
    mesh = plsc.VectorSubcoreMesh(core_axis_name="c", subcore_axis_name="s", num_cores=NC, num_subcores=NS)

    @functools.partial(
        pl.kernel,
        out_type=[
            jax.ShapeDtypeStruct((NC, NPAD), jnp.float32),
            jax.ShapeDtypeStruct((NC, NPAD), jnp.float32),
            jax.ShapeDtypeStruct((NC, NPAD), jnp.float32),
        ],
        mesh=mesh,
        compiler_params=pltpu.CompilerParams(use_tc_tiling_on_sc=False),
        scratch_types=[
            pltpu.VMEM_SHARED((NPAD,), jnp.float32),
            pltpu.VMEM_SHARED((NPAD,), jnp.float32),
            pltpu.VMEM_SHARED((NPAD,), jnp.float32),
            pltpu.VMEM_SHARED((NPAD,), jnp.float32),
            pltpu.VMEM_SHARED((NPAD,), jnp.float32),
            pltpu.VMEM((IB, C), jnp.int32),
            pltpu.VMEM((IB, C), jnp.int32),
            pltpu.VMEM((IB, C), jnp.float32),
            pltpu.VMEM((IB, C), jnp.float32),
            pltpu.VMEM((C,), jnp.float32),
            pltpu.SemaphoreType.DMA,
            pltpu.SemaphoreType.DMA,
        ],
    )
    def pass_a(x0_hbm, x1_hbm, e_hbm, z_hbm, p0_hbm, p1_hbm, pd_hbm,
               tab0_sp, tab1_sp, acc0_sp, acc1_sp, deg_sp,
               sbuf, dbuf, rows0, rows1, ones, semg, sems):
        _edge_body(2, [x0_hbm, x1_hbm], e_hbm, z_hbm,
                   [(acc0_sp, p0_hbm), (acc1_sp, p1_hbm), (deg_sp, pd_hbm)],
                   [tab0_sp, tab1_sp],
                   [(acc0_sp, rows0), (acc1_sp, rows1)], deg_sp,
                   sbuf, dbuf, [rows0, rows1], ones, semg, sems)

    return pass_a


def _make_pass_b():
    mesh = plsc.VectorSubcoreMesh(core_axis_name="c", subcore_axis_name="s", num_cores=NC, num_subcores=NS)

    @functools.partial(
        pl.kernel,
        out_type=jax.ShapeDtypeStruct((NC, NPAD), jnp.float32),
        mesh=mesh,
        compiler_params=pltpu.CompilerParams(use_tc_tiling_on_sc=False),
        scratch_types=[
            pltpu.VMEM_SHARED((NPAD,), jnp.float32),
            pltpu.VMEM_SHARED((NPAD,), jnp.float32),
            pltpu.VMEM((IB, C), jnp.int32),
            pltpu.VMEM((IB, C), jnp.int32),
            pltpu.VMEM((IB, C), jnp.float32),
            pltpu.SemaphoreType.DMA,
            pltpu.SemaphoreType.DMA,
        ],
    )
    def pass_b(z_hbm, e_hbm, zz_hbm, pz_hbm,
               tab_sp, acc_sp, sbuf, dbuf, rows, semg, sems):
        _edge_body(1, [z_hbm], e_hbm, zz_hbm,
                   [(acc_sp, pz_hbm)],
                   [tab_sp],
                   [(acc_sp, rows)], None,
                   sbuf, dbuf, [rows], None, semg, sems)

    return pass_b


_pass_a = _make_pass_a()
_pass_b = _make_pass_b()


def _tc1_body(p0_ref, p1_ref, pd_ref, xT_ref, wl1_ref, bl1_ref, wr1_ref,
              wl2_ref, wr2_ref, bl2_ref, z_ref, r2_ref, deg_ref):
    a0 = p0_ref[0, :] + p0_ref[1, :]
    a1 = p1_ref[0, :] + p1_ref[1, :]
    d = pd_ref[0, :] + pd_ref[1, :]
    dc = jnp.maximum(d, 1.0)
    m0 = a0 / dc
    m1 = a1 / dc
    x0 = xT_ref[0, :]
    x1 = xT_ref[1, :]
    z = jnp.zeros_like(m0)
    r2 = jnp.zeros_like(m0)
    for f in range(16):
        h = jnp.maximum(
            m0 * wl1_ref[f, 0] + m1 * wl1_ref[f, 1] + bl1_ref[f]
            + x0 * wr1_ref[f, 0] + x1 * wr1_ref[f, 1], 0.0)
        z = z + h * wl2_ref[0, f]
        r2 = r2 + h * wr2_ref[0, f]
    z_ref[:] = z
    r2_ref[:] = r2 + bl2_ref[0]
    deg_ref[:] = dc


def _tc1(p0, p1, pd, xT, Wl1, bl1, Wr1, Wl2, Wr2, bl2):
    grid = NPAD // TCB
    smem = pl.BlockSpec(memory_space=pltpu.SMEM)
    vec2 = pl.BlockSpec((NC, TCB), lambda i: (0, i))
    return pl.pallas_call(
        _tc1_body,
        grid=(grid,),
        in_specs=[vec2, vec2, vec2, vec2, smem, smem, smem, smem, smem, smem],
        out_specs=[
            pl.BlockSpec((TCB,), lambda i: (i,)),
            pl.BlockSpec((TCB,), lambda i: (i,)),
            pl.BlockSpec((TCB,), lambda i: (i,)),
        ],
        out_shape=[
            jax.ShapeDtypeStruct((NPAD,), jnp.float32),
            jax.ShapeDtypeStruct((NPAD,), jnp.float32),
            jax.ShapeDtypeStruct((NPAD,), jnp.float32),
        ],
    )(p0, p1, pd, xT, Wl1, bl1, Wr1, Wl2, Wr2, bl2)


def _tc2_body(zp_ref, deg_ref, r2_ref, o_ref):
    zm = (zp_ref[0, :] + zp_ref[1, :]) / deg_ref[:]
    o_ref[:] = jax.nn.sigmoid(zm + r2_ref[:])


def _tc2(zp, degc, r2):
    grid = NPAD // TCB
    return pl.pallas_call(
        _tc2_body,
        grid=(grid,),
        in_specs=[
            pl.BlockSpec((NC, TCB), lambda i: (0, i)),
            pl.BlockSpec((TCB,), lambda i: (i,)),
            pl.BlockSpec((TCB,), lambda i: (i,)),
        ],
        out_specs=pl.BlockSpec((TCB,), lambda i: (i,)),
        out_shape=jax.ShapeDtypeStruct((NPAD,), jnp.float32),
    )(zp, degc, r2)


def kernel(x, edge_index, Wl1, bl1, Wr1, Wl2, bl2, Wr2):
    n = x.shape[0]
    edges3 = edge_index.astype(jnp.int32).reshape(2, NCH, C)
    x0 = x[:, 0]
    x1 = x[:, 1]
    zd = jnp.zeros((NPAD,), jnp.float32)

    p0, p1, pd = _pass_a(x0, x1, edges3, zd)
    xT = jnp.pad(x.T, ((0, 0), (0, NPAD - n)))

    z, r2, degc = _tc1(p0, p1, pd, xT, Wl1, bl1, Wr1, Wl2, Wr2, bl2)

    pz = _pass_b(z, edges3, zd)
    out = _tc2(pz, degc, r2)
    return out[:n]
